# baseline (speedup 1.0000x reference)
import functools

import jax
import jax.numpy as jnp
from jax import lax
from jax.experimental import pallas as pl
from jax.experimental.pallas import tpu as pltpu
from jax.experimental.pallas import tpu_sc as plsc

D = 64
NC = 2
NS = 16
NW = NC * NS
CHUNK = 1024


def _sc_gather(idx2, table2):
    n_chunks = idx2.shape[1]
    per_w = n_chunks * CHUNK
    rows_total = NW * per_w
    mesh = plsc.VectorSubcoreMesh(core_axis_name="c", subcore_axis_name="s")

    @functools.partial(
        pl.kernel,
        mesh=mesh,
        out_type=jax.ShapeDtypeStruct((rows_total, 32), jnp.float32),
        compiler_params=pltpu.CompilerParams(use_tc_tiling_on_sc=False),
        scratch_types=[
            pltpu.VMEM((n_chunks, CHUNK), jnp.int32),
            pltpu.VMEM((CHUNK, 32), jnp.float32),
            pltpu.VMEM((CHUNK, 32), jnp.float32),
            pltpu.SemaphoreType.DMA,
            pltpu.SemaphoreType.DMA,
            pltpu.SemaphoreType.DMA,
            pltpu.SemaphoreType.DMA,
        ],
    )
    def k(idx_hbm, table_hbm, out_hbm, idx_v, rows0, rows1, g0, g1, s0, s1):
        wid = lax.axis_index("s") * NC + lax.axis_index("c")
        base = wid * per_w
        rows = (rows0, rows1)
        gsem = (g0, g1)
        ssem = (s0, s1)

        pltpu.sync_copy(idx_hbm.at[wid], idx_v)

        def gather(g, b):
            return pltpu.make_async_copy(table_hbm.at[idx_v.at[g]], rows[b],
                                         gsem[b])

        def store(g, b):
            return pltpu.make_async_copy(
                rows[b], out_hbm.at[pl.ds(base + g * CHUNK, CHUNK)], ssem[b])

        gather(0, 0).start()

        def pair(j, carry):
            for b in range(2):
                g = 2 * j + b
                gather(g, b).wait()
                @pl.when(g + 1 < n_chunks)
                def _():
                    @pl.when(g >= 1)
                    def _():
                        store(g - 1, 1 - b).wait()
                    gather(g + 1, 1 - b).start()
                store(g, b).start()
            return carry

        lax.fori_loop(0, n_chunks // 2, pair, 0)

        store(n_chunks - 2, 0).wait()
        store(n_chunks - 1, 1).wait()

    return k(idx2, table2)


def kernel(x, table):
    B = x.shape[0] * x.shape[1]
    xf = x.reshape(-1)
    idx2 = (2 * xf[:, None] + jnp.arange(2, dtype=xf.dtype)[None, :])
    idx2 = idx2.reshape(NW, (2 * B) // (NW * CHUNK), CHUNK)
    table2 = table.reshape(2 * table.shape[0], 32)
    out = _sc_gather(idx2, table2)
    return out.reshape(x.shape[0], x.shape[1], D)


# speedup vs baseline: 1.2740x; 1.2740x over previous
import functools

import jax
import jax.numpy as jnp
from jax import lax
from jax.experimental import pallas as pl
from jax.experimental.pallas import tpu as pltpu
from jax.experimental.pallas import tpu_sc as plsc

D = 64
NC = 2
NS = 16
NW = NC * NS
CHUNK = 320


def _sc_gather(xw, tbl):
    n_chunks = xw.shape[1]
    per_w = n_chunks * CHUNK
    B = NW * per_w
    mesh = plsc.VectorSubcoreMesh(core_axis_name="c", subcore_axis_name="s")

    @functools.partial(
        pl.kernel,
        mesh=mesh,
        out_type=jax.ShapeDtypeStruct((B, 128), jnp.float32),
        compiler_params=pltpu.CompilerParams(use_tc_tiling_on_sc=False),
        scratch_types=[
            pltpu.VMEM((n_chunks, CHUNK), jnp.int32),
            pltpu.VMEM((CHUNK, 128), jnp.float32),
            pltpu.VMEM((CHUNK, 128), jnp.float32),
            pltpu.SemaphoreType.DMA,
            pltpu.SemaphoreType.DMA,
            pltpu.SemaphoreType.DMA,
            pltpu.SemaphoreType.DMA,
        ],
    )
    def k(x_hbm, tbl_hbm, out_hbm, idx_v, rows0, rows1, g0, g1, s0, s1):
        wid = lax.axis_index("s") * NC + lax.axis_index("c")
        base = wid * per_w
        rows = (rows0, rows1)
        gsem = (g0, g1)
        ssem = (s0, s1)

        pltpu.sync_copy(x_hbm.at[wid], idx_v)

        def gather(g, b):
            return pltpu.make_async_copy(tbl_hbm.at[idx_v.at[g]], rows[b],
                                         gsem[b])

        def store(g, b):
            return pltpu.make_async_copy(
                rows[b], out_hbm.at[pl.ds(base + g * CHUNK, CHUNK)], ssem[b])

        gather(0, 0).start()

        def pair(j, carry):
            for b in range(2):
                g = 2 * j + b
                gather(g, b).wait()
                @pl.when(g + 1 < n_chunks)
                def _():
                    @pl.when(g >= 1)
                    def _():
                        store(g - 1, 1 - b).wait()
                    gather(g + 1, 1 - b).start()
                store(g, b).start()
            return carry

        lax.fori_loop(0, n_chunks // 2, pair, 0)

        store(n_chunks - 2, 0).wait()
        store(n_chunks - 1, 1).wait()

    return k(xw, tbl)


def kernel(x, table):
    B = x.shape[0] * x.shape[1]
    xw = x.reshape(NW, B // (NW * CHUNK), CHUNK)
    tbl = jnp.pad(table, ((0, 0), (0, 128 - D)))
    out = _sc_gather(xw, tbl)
    return out[:, :D].reshape(x.shape[0], x.shape[1], D)


# speedup vs baseline: 1.3893x; 1.0905x over previous
import functools

import jax
import jax.numpy as jnp
from jax import lax
from jax.experimental import pallas as pl
from jax.experimental.pallas import tpu as pltpu
from jax.experimental.pallas import tpu_sc as plsc

D = 64
NC = 2
NS = 16
NW = NC * NS
CHUNK = 640


def _sc_gather(xw, tbl):
    n_chunks = xw.shape[1]
    per_w = n_chunks * CHUNK
    B = NW * per_w
    mesh = plsc.VectorSubcoreMesh(core_axis_name="c", subcore_axis_name="s")

    @functools.partial(
        pl.kernel,
        mesh=mesh,
        out_type=jax.ShapeDtypeStruct((B, 128), jnp.float32),
        compiler_params=pltpu.CompilerParams(use_tc_tiling_on_sc=False),
        scratch_types=[
            pltpu.VMEM((n_chunks, CHUNK), jnp.int32),
            pltpu.VMEM((CHUNK, D), jnp.float32),
            pltpu.VMEM((CHUNK, D), jnp.float32),
            pltpu.SemaphoreType.DMA,
            pltpu.SemaphoreType.DMA,
            pltpu.SemaphoreType.DMA,
            pltpu.SemaphoreType.DMA,
        ],
    )
    def k(x_hbm, tbl_hbm, out_hbm, idx_v, rows0, rows1, g0, g1, s0, s1):
        wid = lax.axis_index("s") * NC + lax.axis_index("c")
        base = wid * per_w
        rows = (rows0, rows1)
        gsem = (g0, g1)
        ssem = (s0, s1)

        pltpu.sync_copy(x_hbm.at[wid], idx_v)

        def gather(g, b):
            return pltpu.make_async_copy(tbl_hbm.at[idx_v.at[g]], rows[b],
                                         gsem[b])

        def store(g, b):
            return pltpu.make_async_copy(
                rows[b],
                out_hbm.at[pl.ds(base + g * CHUNK, CHUNK), pl.ds(0, D)],
                ssem[b])

        gather(0, 0).start()

        def pair(j, carry):
            for b in range(2):
                g = 2 * j + b
                gather(g, b).wait()
                @pl.when(g + 1 < n_chunks)
                def _():
                    @pl.when(g >= 1)
                    def _():
                        store(g - 1, 1 - b).wait()
                    gather(g + 1, 1 - b).start()
                store(g, b).start()
            return carry

        lax.fori_loop(0, n_chunks // 2, pair, 0)

        store(n_chunks - 2, 0).wait()
        store(n_chunks - 1, 1).wait()

    return k(xw, tbl)


def kernel(x, table):
    B = x.shape[0] * x.shape[1]
    V = table.shape[0]
    xw = x.reshape(NW, B // (NW * CHUNK), CHUNK)
    tbl = jax.lax.optimization_barrier(table.reshape(V // 16, 8, 128))
    tbl = tbl.reshape(V, D)
    out = _sc_gather(xw, tbl)
    return out[:, :D].reshape(x.shape[0], x.shape[1], D)


# speedup vs baseline: 1.4890x; 1.0718x over previous
import functools

import jax
import jax.numpy as jnp
from jax import lax
from jax.experimental import pallas as pl
from jax.experimental.pallas import tpu as pltpu
from jax.experimental.pallas import tpu_sc as plsc

D = 64
NC = 2
NS = 16
NW = NC * NS
CHUNK = 640


def _sc_gather(xw, tbl):
    n_chunks = xw.shape[1]
    per_w = n_chunks * CHUNK
    B = NW * per_w
    mesh = plsc.VectorSubcoreMesh(core_axis_name="c", subcore_axis_name="s")

    @functools.partial(
        pl.kernel,
        mesh=mesh,
        out_type=jax.ShapeDtypeStruct((B, 128), jnp.float32),
        compiler_params=pltpu.CompilerParams(use_tc_tiling_on_sc=False),
        scratch_types=[
            pltpu.VMEM((n_chunks, CHUNK), jnp.int32),
            pltpu.VMEM((CHUNK, D), jnp.float32),
            pltpu.VMEM((CHUNK, D), jnp.float32),
            pltpu.SemaphoreType.DMA,
            pltpu.SemaphoreType.DMA,
            pltpu.SemaphoreType.DMA,
            pltpu.SemaphoreType.DMA,
        ],
    )
    def k(x_hbm, tbl_hbm, out_hbm, idx_v, rows0, rows1, g0, g1, s0, s1):
        wid = lax.axis_index("s") * NC + lax.axis_index("c")
        base = wid * per_w
        rows = (rows0, rows1)
        gsem = (g0, g1)
        ssem = (s0, s1)

        pltpu.sync_copy(x_hbm.at[wid], idx_v)

        def gather(g, b):
            return pltpu.make_async_copy(tbl_hbm.at[idx_v.at[g]], rows[b],
                                         gsem[b])

        def store(g, b):
            return pltpu.make_async_copy(
                rows[b],
                out_hbm.at[pl.ds(base + g * CHUNK, CHUNK), pl.ds(0, D)],
                ssem[b])

        gather(0, 0).start()

        def pair(j, carry):
            for b in range(2):
                g = 2 * j + b
                gather(g, b).wait()
                @pl.when(g + 1 < n_chunks)
                def _():
                    @pl.when(g >= 1)
                    def _():
                        store(g - 1, 1 - b).wait()
                    gather(g + 1, 1 - b).start()
                store(g, b).start()
            return carry

        lax.fori_loop(0, n_chunks // 2, pair, 0)

        store(n_chunks - 2, 0).wait()
        store(n_chunks - 1, 1).wait()

    return k(xw, tbl)


def kernel(x, table):
    B = x.shape[0] * x.shape[1]
    V = table.shape[0]
    xw = (2 * x).reshape(NW, B // (NW * CHUNK), CHUNK)
    tbl = jnp.pad(table, ((0, 0), (0, 128 - D))).reshape(2 * V, D)
    out = _sc_gather(xw, tbl)
    return out[:, :D].reshape(x.shape[0], x.shape[1], D)
